# Initial kernel scaffold; baseline (speedup 1.0000x reference)
import jax
import jax.numpy as jnp
from jax.experimental import pallas as pl


def kernel(x, edge_index, W1l, b1, W1r, W2l, b2, W2r):
    raise NotImplementedError("write your pallas kernel here")


# speedup vs baseline: 4.6821x; 4.6821x over previous
import functools

import jax
import jax.numpy as jnp
from jax import lax
from jax.experimental import pallas as pl
from jax.experimental.pallas import tpu as pltpu
from jax.experimental.pallas import tpu_sc as plsc

N_NODES = 10000
D_FEAT = 128
N_EDGES = 320000

_NC = 2
_NS = 16
_NW = _NC * _NS
_E_PER_TILE = N_EDGES // _NW
_CHUNK = 80
_N_CHUNKS = _E_PER_TILE // _CHUNK
_ROWS_PER_TILE = 624
_TAIL0 = _NS * _ROWS_PER_TILE
_TAIL = N_NODES - _TAIL0

_mesh = plsc.VectorSubcoreMesh(core_axis_name="c", subcore_axis_name="s")


def _zero_acc(s, zf_hbm, acc_sh):
    row0 = s * _ROWS_PER_TILE
    pltpu.sync_copy(zf_hbm.at[pl.ds(row0, _ROWS_PER_TILE)],
                    acc_sh.at[pl.ds(row0, _ROWS_PER_TILE)])

    @pl.when(s == _NS - 1)
    def _tail():
        pltpu.sync_copy(zf_hbm.at[pl.ds(_TAIL0, _TAIL)],
                        acc_sh.at[pl.ds(_TAIL0, _TAIL)])


def _publish_acc(c, s, acc_sh, out_hbm):
    row0 = s * _ROWS_PER_TILE
    pltpu.sync_copy(acc_sh.at[pl.ds(row0, _ROWS_PER_TILE)],
                    out_hbm.at[c, pl.ds(row0, _ROWS_PER_TILE)])

    @pl.when(s == _NS - 1)
    def _tail():
        pltpu.sync_copy(acc_sh.at[pl.ds(_TAIL0, _TAIL)],
                        out_hbm.at[c, pl.ds(_TAIL0, _TAIL)])


def _sc_body(with_count, *refs):
    if with_count:
        (x_hbm, src_hbm, dst_hbm, zf_hbm, ones_hbm,
         agg_out, cnt_out,
         sidx, didx, rows, onesv, acc_sh, sem) = refs
    else:
        (x_hbm, src_hbm, dst_hbm, zf_hbm,
         agg_out,
         sidx, didx, rows, acc_sh, sem) = refs

    c = lax.axis_index("c")
    s = lax.axis_index("s")
    wid = c * _NS + s
    base = wid * _E_PER_TILE

    if with_count:
        pltpu.sync_copy(ones_hbm, onesv)
        _zero_acc(s, zf_hbm, acc_sh)
        plsc.subcore_barrier()

        def cstep(i, carry):
            off = base + i * _CHUNK
            pltpu.sync_copy(dst_hbm.at[pl.ds(off, _CHUNK)], didx)
            pltpu.sync_copy(onesv, acc_sh.at[didx], add=True)
            return carry

        lax.fori_loop(0, _N_CHUNKS, cstep, 0)
        plsc.subcore_barrier()
        _publish_acc(c, s, acc_sh, cnt_out)
        plsc.subcore_barrier()

    _zero_acc(s, zf_hbm, acc_sh)
    plsc.subcore_barrier()

    def step(i, carry):
        off = base + i * _CHUNK
        pltpu.sync_copy(src_hbm.at[pl.ds(off, _CHUNK)], sidx)
        pltpu.sync_copy(dst_hbm.at[pl.ds(off, _CHUNK)], didx)
        pltpu.async_copy(x_hbm.at[sidx], rows, sem).wait()
        pltpu.sync_copy(rows, acc_sh.at[didx], add=True)
        return carry

    lax.fori_loop(0, _N_CHUNKS, step, 0)
    plsc.subcore_barrier()
    _publish_acc(c, s, acc_sh, agg_out)


_sc_pass1 = pl.kernel(
    functools.partial(_sc_body, True),
    out_type=[
        jax.ShapeDtypeStruct((_NC, N_NODES, D_FEAT), jnp.float32),
        jax.ShapeDtypeStruct((_NC, N_NODES, D_FEAT), jnp.float32),
    ],
    mesh=_mesh,
    scratch_types=[
        pltpu.VMEM((_CHUNK,), jnp.int32),
        pltpu.VMEM((_CHUNK,), jnp.int32),
        pltpu.VMEM((_CHUNK, D_FEAT), jnp.float32),
        pltpu.VMEM((_CHUNK, D_FEAT), jnp.float32),
        pltpu.VMEM_SHARED((N_NODES, D_FEAT), jnp.float32),
        pltpu.SemaphoreType.DMA,
    ],
    name="sage_sc_agg_cnt",
)

_sc_pass2 = pl.kernel(
    functools.partial(_sc_body, False),
    out_type=[
        jax.ShapeDtypeStruct((_NC, N_NODES, D_FEAT), jnp.float32),
    ],
    mesh=_mesh,
    scratch_types=[
        pltpu.VMEM((_CHUNK,), jnp.int32),
        pltpu.VMEM((_CHUNK,), jnp.int32),
        pltpu.VMEM((_CHUNK, D_FEAT), jnp.float32),
        pltpu.VMEM_SHARED((N_NODES, D_FEAT), jnp.float32),
        pltpu.SemaphoreType.DMA,
    ],
    name="sage_sc_agg",
)

_BLK = 400
_GRID = N_NODES // _BLK


def _tc_body(relu, agg_ref, cnt_ref, x_ref, wl_ref, wr_ref, b_ref, out_ref):
    aggsum = agg_ref[0] + agg_ref[1]
    cnt = cnt_ref[0, :, :1] + cnt_ref[1, :, :1]
    mean = aggsum / jnp.maximum(cnt, 1.0)
    r = (jnp.dot(mean, wl_ref[...], preferred_element_type=jnp.float32)
         + jnp.dot(x_ref[...], wr_ref[...], preferred_element_type=jnp.float32)
         + b_ref[...])
    out_ref[...] = jnp.maximum(r, 0.0) if relu else r


def _tc_layer(relu):
    return pl.pallas_call(
        functools.partial(_tc_body, relu),
        grid=(_GRID,),
        in_specs=[
            pl.BlockSpec((_NC, _BLK, D_FEAT), lambda i: (0, i, 0)),
            pl.BlockSpec((_NC, _BLK, D_FEAT), lambda i: (0, i, 0)),
            pl.BlockSpec((_BLK, D_FEAT), lambda i: (i, 0)),
            pl.BlockSpec((D_FEAT, D_FEAT), lambda i: (0, 0)),
            pl.BlockSpec((D_FEAT, D_FEAT), lambda i: (0, 0)),
            pl.BlockSpec((1, D_FEAT), lambda i: (0, 0)),
        ],
        out_specs=pl.BlockSpec((_BLK, D_FEAT), lambda i: (i, 0)),
        out_shape=jax.ShapeDtypeStruct((N_NODES, D_FEAT), jnp.float32),
        name="sage_tc_relu" if relu else "sage_tc",
    )


_tc1 = _tc_layer(True)
_tc2 = _tc_layer(False)


def kernel(x, edge_index, W1l, b1, W1r, W2l, b2, W2r):
    src = edge_index[0].astype(jnp.int32)
    dst = edge_index[1].astype(jnp.int32)
    zf = jnp.zeros((N_NODES, D_FEAT), jnp.float32)
    ones = jnp.ones((_CHUNK, D_FEAT), jnp.float32)

    agg1, cnt = _sc_pass1(x, src, dst, zf, ones)
    h = _tc1(agg1, cnt, x, W1l, W1r, b1.reshape(1, D_FEAT))
    (agg2,) = _sc_pass2(h, src, dst, zf)
    out = _tc2(agg2, cnt, h, W2l, W2r, b2.reshape(1, D_FEAT))
    return out


# speedup vs baseline: 9.8957x; 2.1135x over previous
import functools

import jax
import jax.numpy as jnp
from jax import lax
from jax.experimental import pallas as pl
from jax.experimental.pallas import tpu as pltpu
from jax.experimental.pallas import tpu_sc as plsc

N_NODES = 10000
D_FEAT = 128
N_EDGES = 320000

_NC = 2
_NS = 16
_NW = _NC * _NS
_E_PER_TILE = N_EDGES // _NW
_CHUNK = 80
_N_CHUNKS = _E_PER_TILE // _CHUNK
_ROWS_PER_TILE = 624
_TAIL0 = _NS * _ROWS_PER_TILE
_TAIL = N_NODES - _TAIL0

_mesh = plsc.VectorSubcoreMesh(core_axis_name="c", subcore_axis_name="s")


def _zero_acc(s, zf_hbm, acc_sh):
    row0 = s * _ROWS_PER_TILE
    pltpu.sync_copy(zf_hbm.at[pl.ds(row0, _ROWS_PER_TILE)],
                    acc_sh.at[pl.ds(row0, _ROWS_PER_TILE)])

    @pl.when(s == _NS - 1)
    def _tail():
        pltpu.sync_copy(zf_hbm.at[pl.ds(_TAIL0, _TAIL)],
                        acc_sh.at[pl.ds(_TAIL0, _TAIL)])


def _publish_acc(c, s, acc_sh, out_hbm):
    row0 = s * _ROWS_PER_TILE
    pltpu.sync_copy(acc_sh.at[pl.ds(row0, _ROWS_PER_TILE)],
                    out_hbm.at[c, pl.ds(row0, _ROWS_PER_TILE)])

    @pl.when(s == _NS - 1)
    def _tail():
        pltpu.sync_copy(acc_sh.at[pl.ds(_TAIL0, _TAIL)],
                        out_hbm.at[c, pl.ds(_TAIL0, _TAIL)])


_NBUF = 2
_N_GROUPS = _N_CHUNKS // _NBUF
_N_TAIL = _N_CHUNKS - _N_GROUPS * _NBUF
_CGRP = 5
_N_CGROUPS = _N_CHUNKS // _CGRP


def _sc_body(with_count, *refs):
    if with_count:
        (x_hbm, src_hbm, dst_hbm, zf_hbm, ones_hbm,
         agg_out, cnt_out,
         sidx_all, didx_all, acc_sh,
         r0, r1, g0, g1, csem) = refs
    else:
        (x_hbm, src_hbm, dst_hbm, zf_hbm,
         agg_out,
         sidx_all, didx_all, acc_sh,
         r0, r1, g0, g1, csem) = refs

    rows = (r0, r1)
    gsem = (g0, g1)

    c = lax.axis_index("c")
    s = lax.axis_index("s")
    wid = c * _NS + s

    pltpu.sync_copy(src_hbm.at[pl.ds(wid * _E_PER_TILE, _E_PER_TILE)], sidx_all)
    pltpu.sync_copy(dst_hbm.at[wid], didx_all)

    if with_count:
        pltpu.sync_copy(ones_hbm, rows[0])
        _zero_acc(s, zf_hbm, acc_sh)
        plsc.subcore_barrier()

        def cstep(g, carry):
            for b in range(_CGRP):
                pltpu.async_copy(rows[0], acc_sh.at[didx_all.at[g * _CGRP + b]],
                                 csem, add=True)
            for b in range(_CGRP):
                pltpu.make_async_copy(
                    rows[0], acc_sh.at[didx_all.at[g * _CGRP + b]], csem).wait()
            return carry

        lax.fori_loop(0, _N_CGROUPS, cstep, 0)
        plsc.subcore_barrier()
        _publish_acc(c, s, acc_sh, cnt_out)
        plsc.subcore_barrier()

    _zero_acc(s, zf_hbm, acc_sh)
    plsc.subcore_barrier()

    def chunk_step(i, b):
        pltpu.make_async_copy(x_hbm.at[sidx_all.at[pl.ds(i * _CHUNK, _CHUNK)]], rows[b],
                              gsem[b]).wait()
        pltpu.sync_copy(rows[b], acc_sh.at[didx_all.at[i]], add=True)

        @pl.when(i + _NBUF < _N_CHUNKS)
        def _prefetch():
            pltpu.async_copy(x_hbm.at[sidx_all.at[pl.ds((i + _NBUF) * _CHUNK, _CHUNK)]],
                             rows[b], gsem[b])

    for b in range(_NBUF):
        pltpu.async_copy(x_hbm.at[sidx_all.at[pl.ds(b * _CHUNK, _CHUNK)]], rows[b], gsem[b])

    def step(g, carry):
        for b in range(_NBUF):
            chunk_step(g * _NBUF + b, b)
        return carry

    lax.fori_loop(0, _N_GROUPS, step, 0)
    for t in range(_N_TAIL):
        chunk_step(_N_GROUPS * _NBUF + t, t)

    plsc.subcore_barrier()
    _publish_acc(c, s, acc_sh, agg_out)


_SCRATCH = [
    pltpu.VMEM((_E_PER_TILE,), jnp.int32),
    pltpu.VMEM((_N_CHUNKS, _CHUNK), jnp.int32),
    pltpu.VMEM_SHARED((N_NODES, D_FEAT), jnp.float32),
] + [pltpu.VMEM((_CHUNK, D_FEAT), jnp.float32)] * _NBUF \
  + [pltpu.SemaphoreType.DMA] * (_NBUF + 1)

_sc_pass1 = pl.kernel(
    functools.partial(_sc_body, True),
    out_type=[
        jax.ShapeDtypeStruct((_NC, N_NODES, D_FEAT), jnp.float32),
        jax.ShapeDtypeStruct((_NC, N_NODES, D_FEAT), jnp.float32),
    ],
    mesh=_mesh,
    scratch_types=_SCRATCH,
    name="sage_sc_agg_cnt",
)

_sc_pass2 = pl.kernel(
    functools.partial(_sc_body, False),
    out_type=[
        jax.ShapeDtypeStruct((_NC, N_NODES, D_FEAT), jnp.float32),
    ],
    mesh=_mesh,
    scratch_types=_SCRATCH,
    name="sage_sc_agg",
)

_BLK = 400
_GRID = N_NODES // _BLK


def _tc_body(relu, agg_ref, cnt_ref, x_ref, wl_ref, wr_ref, b_ref, out_ref):
    aggsum = agg_ref[0] + agg_ref[1]
    cnt = cnt_ref[0, :, :1] + cnt_ref[1, :, :1]
    mean = aggsum / jnp.maximum(cnt, 1.0)
    r = (jnp.dot(mean, wl_ref[...], preferred_element_type=jnp.float32)
         + jnp.dot(x_ref[...], wr_ref[...], preferred_element_type=jnp.float32)
         + b_ref[...])
    out_ref[...] = jnp.maximum(r, 0.0) if relu else r


def _tc_layer(relu):
    return pl.pallas_call(
        functools.partial(_tc_body, relu),
        grid=(_GRID,),
        in_specs=[
            pl.BlockSpec((_NC, _BLK, D_FEAT), lambda i: (0, i, 0)),
            pl.BlockSpec((_NC, _BLK, D_FEAT), lambda i: (0, i, 0)),
            pl.BlockSpec((_BLK, D_FEAT), lambda i: (i, 0)),
            pl.BlockSpec((D_FEAT, D_FEAT), lambda i: (0, 0)),
            pl.BlockSpec((D_FEAT, D_FEAT), lambda i: (0, 0)),
            pl.BlockSpec((1, D_FEAT), lambda i: (0, 0)),
        ],
        out_specs=pl.BlockSpec((_BLK, D_FEAT), lambda i: (i, 0)),
        out_shape=jax.ShapeDtypeStruct((N_NODES, D_FEAT), jnp.float32),
        name="sage_tc_relu" if relu else "sage_tc",
    )


_tc1 = _tc_layer(True)
_tc2 = _tc_layer(False)


def kernel(x, edge_index, W1l, b1, W1r, W2l, b2, W2r):
    src = edge_index[0].astype(jnp.int32)
    dst = edge_index[1].astype(jnp.int32).reshape(_NW, _N_CHUNKS, _CHUNK)
    zf = jnp.zeros((N_NODES, D_FEAT), jnp.float32)
    ones = jnp.ones((_CHUNK, D_FEAT), jnp.float32)

    agg1, cnt = _sc_pass1(x, src, dst, zf, ones)
    h = _tc1(agg1, cnt, x, W1l, W1r, b1.reshape(1, D_FEAT))
    (agg2,) = _sc_pass2(h, src, dst, zf)
    out = _tc2(agg2, cnt, h, W2l, W2r, b2.reshape(1, D_FEAT))
    return out


# speedup vs baseline: 11.0255x; 1.1142x over previous
import functools

import jax
import jax.numpy as jnp
from jax import lax
from jax.experimental import pallas as pl
from jax.experimental.pallas import tpu as pltpu
from jax.experimental.pallas import tpu_sc as plsc

N_NODES = 10000
D_FEAT = 128
N_EDGES = 320000

_NC = 2
_NS = 16
_NW = _NC * _NS
_E_PER_TILE = N_EDGES // _NW
_CHUNK = 80
_N_CHUNKS = _E_PER_TILE // _CHUNK
_ROWS_PER_TILE = 624
_TAIL0 = _NS * _ROWS_PER_TILE
_TAIL = N_NODES - _TAIL0

_mesh = plsc.VectorSubcoreMesh(core_axis_name="c", subcore_axis_name="s")


def _zero_acc(s, zf_hbm, acc_sh):
    row0 = s * _ROWS_PER_TILE
    pltpu.sync_copy(zf_hbm.at[pl.ds(row0, _ROWS_PER_TILE)],
                    acc_sh.at[pl.ds(row0, _ROWS_PER_TILE)])

    @pl.when(s == _NS - 1)
    def _tail():
        pltpu.sync_copy(zf_hbm.at[pl.ds(_TAIL0, _TAIL)],
                        acc_sh.at[pl.ds(_TAIL0, _TAIL)])


def _publish_acc(c, s, acc_sh, out_hbm):
    row0 = s * _ROWS_PER_TILE
    pltpu.sync_copy(acc_sh.at[pl.ds(row0, _ROWS_PER_TILE)],
                    out_hbm.at[c, pl.ds(row0, _ROWS_PER_TILE)])

    @pl.when(s == _NS - 1)
    def _tail():
        pltpu.sync_copy(acc_sh.at[pl.ds(_TAIL0, _TAIL)],
                        out_hbm.at[c, pl.ds(_TAIL0, _TAIL)])


_NBUF = 3
_N_GROUPS = _N_CHUNKS // _NBUF
_N_TAIL = _N_CHUNKS - _N_GROUPS * _NBUF
_CGRP = 5
_N_CGROUPS = _N_CHUNKS // _CGRP


def _sc_body(with_count, *refs):
    if with_count:
        (x_hbm, src_hbm, dst_hbm, zf_hbm, ones_hbm,
         agg_out, cnt_out,
         sidx_all, acc_sh,
         r0, r1, r2, d0, d1, d2,
         g0, g1, g2, s0, s1, s2, ds0, ds1, ds2) = refs
    else:
        (x_hbm, src_hbm, dst_hbm, zf_hbm,
         agg_out,
         sidx_all, acc_sh,
         r0, r1, r2, d0, d1, d2,
         g0, g1, g2, s0, s1, s2, ds0, ds1, ds2) = refs

    rows = (r0, r1, r2)
    dstage = (d0, d1, d2)
    gsem = (g0, g1, g2)
    ssem = (s0, s1, s2)
    dsem = (ds0, ds1, ds2)

    c = lax.axis_index("c")
    s = lax.axis_index("s")
    wid = c * _NS + s
    ebase = wid * _E_PER_TILE

    pltpu.sync_copy(src_hbm.at[pl.ds(ebase, _E_PER_TILE)], sidx_all)

    def load_dstage(i, b):
        pltpu.async_copy(dst_hbm.at[pl.ds(ebase + i * _CHUNK, _CHUNK)],
                         dstage[b].at[0], dsem[b])

    def wait_dstage(i, b):
        pltpu.make_async_copy(dst_hbm.at[pl.ds(ebase + i * _CHUNK, _CHUNK)],
                              dstage[b].at[0], dsem[b]).wait()

    def gather(i, b):
        pltpu.async_copy(x_hbm.at[sidx_all.at[pl.ds(i * _CHUNK, _CHUNK)]],
                         rows[b], gsem[b])

    def wait_gather(i, b):
        pltpu.make_async_copy(x_hbm.at[sidx_all.at[pl.ds(i * _CHUNK, _CHUNK)]],
                              rows[b], gsem[b]).wait()

    def fire_scatter(src_buf, b):
        pltpu.async_copy(src_buf, acc_sh.at[dstage[b].at[0]], ssem[b],
                         add=True)

    def drain_scatter(src_buf, b):
        pltpu.make_async_copy(src_buf, acc_sh.at[dstage[b].at[0]],
                              ssem[b]).wait()

    def chunk_step(i, b, do_gather, first, last):
        p = (b + 2) % _NBUF
        wait_dstage(i, b)
        if do_gather:
            wait_gather(i, b)
            fire_scatter(rows[b], b)
        else:
            fire_scatter(rows[0], b)
        if not first:
            drain_scatter(rows[p] if do_gather else rows[0], p)
        if not last:
            load_dstage(i + 2, p)
            if do_gather:
                gather(i + 2, p)

    def run_stage(do_gather):
        for b in range(2):
            load_dstage(b, b)
            if do_gather:
                gather(b, b)
        for b in range(_NBUF):
            chunk_step(b, b, do_gather, b == 0, False)

        def step(g, carry):
            for b in range(_NBUF):
                chunk_step(g * _NBUF + b, b, do_gather, False, False)
            return carry

        lax.fori_loop(1, _N_GROUPS, step, 0)
        for t in range(_N_TAIL):
            chunk_step(_N_GROUPS * _NBUF + t, t, do_gather, False, True)
        last_slot = (_N_TAIL - 1) if _N_TAIL else (_NBUF - 1)
        drain_scatter(rows[last_slot] if do_gather else rows[0], last_slot)

    if with_count:
        pltpu.sync_copy(ones_hbm, rows[0])
        _zero_acc(s, zf_hbm, acc_sh)
        plsc.subcore_barrier()
        run_stage(do_gather=False)
        plsc.subcore_barrier()
        _publish_acc(c, s, acc_sh, cnt_out)
        plsc.subcore_barrier()

    _zero_acc(s, zf_hbm, acc_sh)
    plsc.subcore_barrier()
    run_stage(do_gather=True)
    plsc.subcore_barrier()
    _publish_acc(c, s, acc_sh, agg_out)


_SCRATCH = [
    pltpu.VMEM((_E_PER_TILE,), jnp.int32),
    pltpu.VMEM_SHARED((N_NODES, D_FEAT), jnp.float32),
] + [pltpu.VMEM((_CHUNK, D_FEAT), jnp.float32)] * _NBUF \
  + [pltpu.VMEM((8, _CHUNK), jnp.int32)] * _NBUF \
  + [pltpu.SemaphoreType.DMA] * (3 * _NBUF)

_sc_pass1 = pl.kernel(
    functools.partial(_sc_body, True),
    out_type=[
        jax.ShapeDtypeStruct((_NC, N_NODES, D_FEAT), jnp.float32),
        jax.ShapeDtypeStruct((_NC, N_NODES, D_FEAT), jnp.float32),
    ],
    mesh=_mesh,
    scratch_types=_SCRATCH,
    name="sage_sc_agg_cnt",
)

_sc_pass2 = pl.kernel(
    functools.partial(_sc_body, False),
    out_type=[
        jax.ShapeDtypeStruct((_NC, N_NODES, D_FEAT), jnp.float32),
    ],
    mesh=_mesh,
    scratch_types=_SCRATCH,
    name="sage_sc_agg",
)

_BLK = 400
_GRID = N_NODES // _BLK


def _tc_body(relu, agg_ref, cnt_ref, x_ref, wl_ref, wr_ref, b_ref, out_ref):
    aggsum = agg_ref[0] + agg_ref[1]
    cnt = cnt_ref[0, :, :1] + cnt_ref[1, :, :1]
    mean = aggsum / jnp.maximum(cnt, 1.0)
    r = (jnp.dot(mean, wl_ref[...], preferred_element_type=jnp.float32)
         + jnp.dot(x_ref[...], wr_ref[...], preferred_element_type=jnp.float32)
         + b_ref[...])
    out_ref[...] = jnp.maximum(r, 0.0) if relu else r


def _tc_layer(relu):
    return pl.pallas_call(
        functools.partial(_tc_body, relu),
        grid=(_GRID,),
        in_specs=[
            pl.BlockSpec((_NC, _BLK, D_FEAT), lambda i: (0, i, 0)),
            pl.BlockSpec((_NC, _BLK, D_FEAT), lambda i: (0, i, 0)),
            pl.BlockSpec((_BLK, D_FEAT), lambda i: (i, 0)),
            pl.BlockSpec((D_FEAT, D_FEAT), lambda i: (0, 0)),
            pl.BlockSpec((D_FEAT, D_FEAT), lambda i: (0, 0)),
            pl.BlockSpec((1, D_FEAT), lambda i: (0, 0)),
        ],
        out_specs=pl.BlockSpec((_BLK, D_FEAT), lambda i: (i, 0)),
        out_shape=jax.ShapeDtypeStruct((N_NODES, D_FEAT), jnp.float32),
        name="sage_tc_relu" if relu else "sage_tc",
    )


_tc1 = _tc_layer(True)
_tc2 = _tc_layer(False)


def kernel(x, edge_index, W1l, b1, W1r, W2l, b2, W2r):
    src = edge_index[0].astype(jnp.int32)
    dst = edge_index[1].astype(jnp.int32)
    zf = jnp.zeros((N_NODES, D_FEAT), jnp.float32)
    ones = jnp.ones((_CHUNK, D_FEAT), jnp.float32)

    agg1, cnt = _sc_pass1(x, src, dst, zf, ones)
    h = _tc1(agg1, cnt, x, W1l, W1r, b1.reshape(1, D_FEAT))
    (agg2,) = _sc_pass2(h, src, dst, zf)
    out = _tc2(agg2, cnt, h, W2l, W2r, b2.reshape(1, D_FEAT))
    return out


# speedup vs baseline: 11.0494x; 1.0022x over previous
import functools

import jax
import jax.numpy as jnp
from jax import lax
from jax.experimental import pallas as pl
from jax.experimental.pallas import tpu as pltpu
from jax.experimental.pallas import tpu_sc as plsc

N_NODES = 10000
D_FEAT = 128
N_EDGES = 320000

_NC = 2
_NS = 16
_NW = _NC * _NS
_E_PER_TILE = N_EDGES // _NW
_CHUNK = 80
_N_CHUNKS = _E_PER_TILE // _CHUNK
_ROWS_PER_TILE = 624
_TAIL0 = _NS * _ROWS_PER_TILE
_TAIL = N_NODES - _TAIL0

_mesh = plsc.VectorSubcoreMesh(core_axis_name="c", subcore_axis_name="s")


def _zero_acc(s, zf_hbm, acc_sh):
    row0 = s * _ROWS_PER_TILE
    pltpu.sync_copy(zf_hbm.at[pl.ds(row0, _ROWS_PER_TILE)],
                    acc_sh.at[pl.ds(row0, _ROWS_PER_TILE)])

    @pl.when(s == _NS - 1)
    def _tail():
        pltpu.sync_copy(zf_hbm.at[pl.ds(_TAIL0, _TAIL)],
                        acc_sh.at[pl.ds(_TAIL0, _TAIL)])


def _publish_acc(c, s, acc_sh, out_hbm):
    row0 = s * _ROWS_PER_TILE
    pltpu.sync_copy(acc_sh.at[pl.ds(row0, _ROWS_PER_TILE)],
                    out_hbm.at[c, pl.ds(row0, _ROWS_PER_TILE)])

    @pl.when(s == _NS - 1)
    def _tail():
        pltpu.sync_copy(acc_sh.at[pl.ds(_TAIL0, _TAIL)],
                        out_hbm.at[c, pl.ds(_TAIL0, _TAIL)])


def _sc_body(with_count, nbuf, *refs):
    if with_count:
        (x_hbm, src_hbm, dst_hbm, zf_hbm, ones_hbm,
         agg_out, cnt_out) = refs[:7]
        rest = refs[7:]
    else:
        (x_hbm, src_hbm, dst_hbm, zf_hbm, agg_out) = refs[:5]
        rest = refs[5:]
    acc_sh = rest[0]
    rows = rest[1:1 + nbuf]
    dstage = rest[1 + nbuf:1 + 2 * nbuf]
    sstage = rest[1 + 2 * nbuf:1 + 3 * nbuf]
    gsem = rest[1 + 3 * nbuf:1 + 4 * nbuf]
    ssem = rest[1 + 4 * nbuf:1 + 5 * nbuf]
    dsem = rest[1 + 5 * nbuf:1 + 6 * nbuf]
    xsem = rest[1 + 6 * nbuf:1 + 7 * nbuf]

    c = lax.axis_index("c")
    s = lax.axis_index("s")
    wid = c * _NS + s
    ebase = wid * _E_PER_TILE
    row0 = s * _ROWS_PER_TILE

    def load_idx(hbm, i, buf, sem):
        pltpu.async_copy(hbm.at[pl.ds(ebase + i * _CHUNK, _CHUNK)],
                         buf.at[0], sem)

    def wait_idx(hbm, i, buf, sem):
        pltpu.make_async_copy(hbm.at[pl.ds(ebase + i * _CHUNK, _CHUNK)],
                              buf.at[0], sem).wait()

    def gather(i, b):
        pltpu.async_copy(x_hbm.at[sstage[b].at[0]], rows[b], gsem[b])

    def wait_gather(b):
        pltpu.make_async_copy(x_hbm.at[sstage[b].at[0]], rows[b],
                              gsem[b]).wait()

    def fire_scatter(acc, src_buf, b):
        pltpu.async_copy(src_buf, acc.at[dstage[b].at[0]], ssem[b], add=True)

    def drain_scatter(acc, src_buf, b):
        pltpu.make_async_copy(src_buf, acc.at[dstage[b].at[0]],
                              ssem[b]).wait()

    def chunk_step(i, b, do_gather, acc, first, last):
        p = (b + nbuf - 1) % nbuf
        wait_idx(dst_hbm, i, dstage[b], dsem[b])
        if do_gather:
            wait_gather(b)
            fire_scatter(acc, rows[b], b)
        else:
            fire_scatter(acc, rows[0], b)
        if not first:
            drain_scatter(acc, rows[p] if do_gather else rows[0], p)
        if not last:
            load_idx(dst_hbm, i + nbuf - 1, dstage[p], dsem[p])
            if do_gather:
                wait_idx(src_hbm, i + nbuf - 1, sstage[p], xsem[p])
                gather(i + nbuf - 1, p)

                @pl.when(i + nbuf < _N_CHUNKS)
                def _stage_next_sidx():
                    load_idx(src_hbm, i + nbuf, sstage[b], xsem[b])

    def run_stage(do_gather, acc):
        for b in range(nbuf - 1):
            load_idx(dst_hbm, b, dstage[b], dsem[b])
            if do_gather:
                load_idx(src_hbm, b, sstage[b], xsem[b])
        if do_gather:
            load_idx(src_hbm, nbuf - 1, sstage[nbuf - 1], xsem[nbuf - 1])
            for b in range(nbuf - 1):
                wait_idx(src_hbm, b, sstage[b], xsem[b])
                gather(b, b)
        for b in range(nbuf):
            chunk_step(b, b, do_gather, acc, b == 0, False)

        def step(g, carry):
            for b in range(nbuf):
                chunk_step(g * nbuf + b, b, do_gather, acc, False, False)
            return carry

        n_groups = _N_CHUNKS // nbuf
        n_tail = _N_CHUNKS - n_groups * nbuf
        lax.fori_loop(1, n_groups, step, 0)
        for t in range(n_tail):
            chunk_step(n_groups * nbuf + t, t, do_gather, acc, False, True)
        last_slot = (n_tail - 1) if n_tail else (nbuf - 1)
        drain_scatter(acc, rows[last_slot] if do_gather else rows[0],
                      last_slot)

    if with_count:
        pltpu.sync_copy(ones_hbm, rows[0])
        _zero_acc(s, zf_hbm, acc_sh)
        plsc.subcore_barrier()
        run_stage(False, acc_sh)
        plsc.subcore_barrier()
        _publish_acc(c, s, acc_sh, cnt_out)
        plsc.subcore_barrier()

    _zero_acc(s, zf_hbm, acc_sh)
    plsc.subcore_barrier()
    run_stage(True, acc_sh)
    plsc.subcore_barrier()
    _publish_acc(c, s, acc_sh, agg_out)


def _scratch(nbuf):
    sc = [pltpu.VMEM_SHARED((N_NODES, D_FEAT), jnp.float32)]
    sc += [pltpu.VMEM((_CHUNK, D_FEAT), jnp.float32)] * nbuf
    sc += [pltpu.VMEM((8, _CHUNK), jnp.int32)] * (2 * nbuf)
    sc += [pltpu.SemaphoreType.DMA] * (4 * nbuf)
    return sc


_sc_pass1 = pl.kernel(
    functools.partial(_sc_body, True, 3),
    out_type=[
        jax.ShapeDtypeStruct((_NC, N_NODES, D_FEAT), jnp.float32),
        jax.ShapeDtypeStruct((_NC, N_NODES, D_FEAT), jnp.float32),
    ],
    mesh=_mesh,
    scratch_types=_scratch(3),
    name="sage_sc_agg_cnt",
)

_sc_pass2 = pl.kernel(
    functools.partial(_sc_body, False, 3),
    out_type=[
        jax.ShapeDtypeStruct((_NC, N_NODES, D_FEAT), jnp.float32),
    ],
    mesh=_mesh,
    scratch_types=_scratch(3),
    name="sage_sc_agg",
)

_BLK = 400
_GRID = N_NODES // _BLK


def _tc_body(relu, agg_ref, cnt_ref, x_ref, wl_ref, wr_ref, b_ref, out_ref):
    aggsum = agg_ref[0] + agg_ref[1]
    cnt = cnt_ref[0, :, :1] + cnt_ref[1, :, :1]
    mean = aggsum / jnp.maximum(cnt, 1.0)
    r = (jnp.dot(mean, wl_ref[...], preferred_element_type=jnp.float32)
         + jnp.dot(x_ref[...], wr_ref[...], preferred_element_type=jnp.float32)
         + b_ref[...])
    out_ref[...] = jnp.maximum(r, 0.0) if relu else r


def _tc_layer(relu):
    return pl.pallas_call(
        functools.partial(_tc_body, relu),
        grid=(_GRID,),
        in_specs=[
            pl.BlockSpec((_NC, _BLK, D_FEAT), lambda i: (0, i, 0)),
            pl.BlockSpec((_NC, _BLK, D_FEAT), lambda i: (0, i, 0)),
            pl.BlockSpec((_BLK, D_FEAT), lambda i: (i, 0)),
            pl.BlockSpec((D_FEAT, D_FEAT), lambda i: (0, 0)),
            pl.BlockSpec((D_FEAT, D_FEAT), lambda i: (0, 0)),
            pl.BlockSpec((1, D_FEAT), lambda i: (0, 0)),
        ],
        out_specs=pl.BlockSpec((_BLK, D_FEAT), lambda i: (i, 0)),
        out_shape=jax.ShapeDtypeStruct((N_NODES, D_FEAT), jnp.float32),
        name="sage_tc_relu" if relu else "sage_tc",
    )


_tc1 = _tc_layer(True)
_tc2 = _tc_layer(False)


def kernel(x, edge_index, W1l, b1, W1r, W2l, b2, W2r):
    src = edge_index[0].astype(jnp.int32)
    dst = edge_index[1].astype(jnp.int32)
    zf = jnp.zeros((N_NODES, D_FEAT), jnp.float32)
    ones = jnp.ones((_CHUNK, D_FEAT), jnp.float32)

    agg1, cnt = _sc_pass1(x, src, dst, zf, ones)
    h = _tc1(agg1, cnt, x, W1l, W1r, b1.reshape(1, D_FEAT))
    (agg2,) = _sc_pass2(h, src, dst, zf)
    out = _tc2(agg2, cnt, h, W2l, W2r, b2.reshape(1, D_FEAT))
    return out


# speedup vs baseline: 11.7876x; 1.0668x over previous
import functools

import jax
import jax.numpy as jnp
from jax import lax
from jax.experimental import pallas as pl
from jax.experimental.pallas import tpu as pltpu
from jax.experimental.pallas import tpu_sc as plsc

N_NODES = 10000
D_FEAT = 128
N_EDGES = 320000

_NC = 2
_NS = 16
_NW = _NC * _NS
_E_PER_TILE = N_EDGES // _NW
_CHUNK = 80
_N_CHUNKS = _E_PER_TILE // _CHUNK
_ROWS_PER_TILE = 624
_TAIL0 = _NS * _ROWS_PER_TILE
_TAIL = N_NODES - _TAIL0

_mesh = plsc.VectorSubcoreMesh(core_axis_name="c", subcore_axis_name="s")


def _zero_acc(s, zf_hbm, acc_sh):
    row0 = s * _ROWS_PER_TILE
    pltpu.sync_copy(zf_hbm.at[pl.ds(row0, _ROWS_PER_TILE)],
                    acc_sh.at[pl.ds(row0, _ROWS_PER_TILE)])

    @pl.when(s == _NS - 1)
    def _tail():
        pltpu.sync_copy(zf_hbm.at[pl.ds(_TAIL0, _TAIL)],
                        acc_sh.at[pl.ds(_TAIL0, _TAIL)])


def _publish_acc(c, s, acc_sh, out_hbm):
    row0 = s * _ROWS_PER_TILE
    pltpu.sync_copy(acc_sh.at[pl.ds(row0, _ROWS_PER_TILE)],
                    out_hbm.at[c, pl.ds(row0, _ROWS_PER_TILE)])

    @pl.when(s == _NS - 1)
    def _tail():
        pltpu.sync_copy(acc_sh.at[pl.ds(_TAIL0, _TAIL)],
                        out_hbm.at[c, pl.ds(_TAIL0, _TAIL)])


def _sc_body(with_count, nbuf, *refs):
    if with_count:
        (x_hbm, src_hbm, dst_hbm, zf_hbm, ones_hbm,
         agg_out, cnt_out) = refs[:7]
        rest = refs[7:]
    else:
        (x_hbm, src_hbm, dst_hbm, zf_hbm, agg_out) = refs[:5]
        rest = refs[5:]
    acc_sh = rest[0]
    rows = rest[1:1 + nbuf]
    dstage = rest[1 + nbuf:1 + 2 * nbuf]
    sstage = rest[1 + 2 * nbuf:1 + 3 * nbuf]
    gsem = rest[1 + 3 * nbuf:1 + 4 * nbuf]
    ssem = rest[1 + 4 * nbuf:1 + 5 * nbuf]
    dsem = rest[1 + 5 * nbuf:1 + 6 * nbuf]
    xsem = rest[1 + 6 * nbuf:1 + 7 * nbuf]

    c = lax.axis_index("c")
    s = lax.axis_index("s")
    wid = c * _NS + s
    ebase = wid * _E_PER_TILE
    row0 = s * _ROWS_PER_TILE

    def load_idx(hbm, i, buf, sem):
        pltpu.async_copy(hbm.at[pl.ds(ebase + i * _CHUNK, _CHUNK)],
                         buf.at[0], sem)

    def wait_idx(hbm, i, buf, sem):
        pltpu.make_async_copy(hbm.at[pl.ds(ebase + i * _CHUNK, _CHUNK)],
                              buf.at[0], sem).wait()

    def gather(i, b):
        pltpu.async_copy(x_hbm.at[sstage[b].at[0]], rows[b], gsem[b])

    def wait_gather(b):
        pltpu.make_async_copy(x_hbm.at[sstage[b].at[0]], rows[b],
                              gsem[b]).wait()

    def fire_scatter(acc, src_buf, b):
        pltpu.async_copy(src_buf, acc.at[dstage[b].at[0]], ssem[b], add=True)

    def drain_scatter(acc, src_buf, b):
        pltpu.make_async_copy(src_buf, acc.at[dstage[b].at[0]],
                              ssem[b]).wait()

    def chunk_step(i, b, do_gather, acc, first, last):
        p = (b + nbuf - 1) % nbuf
        wait_idx(dst_hbm, i, dstage[b], dsem[b])
        if do_gather:
            wait_gather(b)
            fire_scatter(acc, rows[b], b)
        else:
            fire_scatter(acc, rows[0], b)
        if not first:
            drain_scatter(acc, rows[p] if do_gather else rows[0], p)
        if not last:
            load_idx(dst_hbm, i + nbuf - 1, dstage[p], dsem[p])
            if do_gather:
                wait_idx(src_hbm, i + nbuf - 1, sstage[p], xsem[p])
                gather(i + nbuf - 1, p)

                @pl.when(i + nbuf < _N_CHUNKS)
                def _stage_next_sidx():
                    load_idx(src_hbm, i + nbuf, sstage[b], xsem[b])

    def run_stage(do_gather, acc):
        for b in range(nbuf - 1):
            load_idx(dst_hbm, b, dstage[b], dsem[b])
            if do_gather:
                load_idx(src_hbm, b, sstage[b], xsem[b])
        if do_gather:
            load_idx(src_hbm, nbuf - 1, sstage[nbuf - 1], xsem[nbuf - 1])
            for b in range(nbuf - 1):
                wait_idx(src_hbm, b, sstage[b], xsem[b])
                gather(b, b)
        for b in range(nbuf):
            chunk_step(b, b, do_gather, acc, b == 0, False)

        def step(g, carry):
            for b in range(nbuf):
                chunk_step(g * nbuf + b, b, do_gather, acc, False, False)
            return carry

        n_groups = _N_CHUNKS // nbuf
        n_tail = _N_CHUNKS - n_groups * nbuf
        lax.fori_loop(1, n_groups, step, 0)
        for t in range(n_tail):
            chunk_step(n_groups * nbuf + t, t, do_gather, acc, False, True)
        last_slot = (n_tail - 1) if n_tail else (nbuf - 1)
        drain_scatter(acc, rows[last_slot] if do_gather else rows[0],
                      last_slot)

    if with_count:
        pltpu.sync_copy(ones_hbm, rows[0])
        _zero_acc(s, zf_hbm, acc_sh)
        plsc.subcore_barrier()
        run_stage(False, acc_sh)
        plsc.subcore_barrier()
        _publish_acc(c, s, acc_sh, cnt_out)
        plsc.subcore_barrier()

    _zero_acc(s, zf_hbm, acc_sh)
    plsc.subcore_barrier()
    run_stage(True, acc_sh)
    plsc.subcore_barrier()
    _publish_acc(c, s, acc_sh, agg_out)


def _scratch(nbuf):
    sc = [pltpu.VMEM_SHARED((N_NODES, D_FEAT), jnp.float32)]
    sc += [pltpu.VMEM((_CHUNK, D_FEAT), jnp.float32)] * nbuf
    sc += [pltpu.VMEM((8, _CHUNK), jnp.int32)] * (2 * nbuf)
    sc += [pltpu.SemaphoreType.DMA] * (4 * nbuf)
    return sc


_sc_pass1 = pl.kernel(
    functools.partial(_sc_body, True, 3),
    out_type=[
        jax.ShapeDtypeStruct((_NC, N_NODES, D_FEAT), jnp.float32),
        jax.ShapeDtypeStruct((_NC, N_NODES, D_FEAT), jnp.float32),
    ],
    mesh=_mesh,
    scratch_types=_scratch(3),
    name="sage_sc_agg_cnt",
)

_sc_pass2 = pl.kernel(
    functools.partial(_sc_body, False, 3),
    out_type=[
        jax.ShapeDtypeStruct((_NC, N_NODES, D_FEAT), jnp.float32),
    ],
    mesh=_mesh,
    scratch_types=_scratch(3),
    name="sage_sc_agg",
)

_BLK = 2000
_GRID = N_NODES // _BLK


def _tc_body(relu, agg_ref, cnt_ref, x_ref, wl_ref, wr_ref, b_ref, out_ref):
    aggsum = agg_ref[0] + agg_ref[1]
    cnt = cnt_ref[0, :, :1] + cnt_ref[1, :, :1]
    mean = aggsum / jnp.maximum(cnt, 1.0)
    r = (jnp.dot(mean, wl_ref[...], preferred_element_type=jnp.float32)
         + jnp.dot(x_ref[...], wr_ref[...], preferred_element_type=jnp.float32)
         + b_ref[...])
    out_ref[...] = jnp.maximum(r, 0.0) if relu else r


def _tc_layer(relu):
    return pl.pallas_call(
        functools.partial(_tc_body, relu),
        grid=(_GRID,),
        in_specs=[
            pl.BlockSpec((_NC, _BLK, D_FEAT), lambda i: (0, i, 0)),
            pl.BlockSpec((_NC, _BLK, D_FEAT), lambda i: (0, i, 0)),
            pl.BlockSpec((_BLK, D_FEAT), lambda i: (i, 0)),
            pl.BlockSpec((D_FEAT, D_FEAT), lambda i: (0, 0)),
            pl.BlockSpec((D_FEAT, D_FEAT), lambda i: (0, 0)),
            pl.BlockSpec((1, D_FEAT), lambda i: (0, 0)),
        ],
        out_specs=pl.BlockSpec((_BLK, D_FEAT), lambda i: (i, 0)),
        out_shape=jax.ShapeDtypeStruct((N_NODES, D_FEAT), jnp.float32),
        name="sage_tc_relu" if relu else "sage_tc",
    )


_tc1 = _tc_layer(True)
_tc2 = _tc_layer(False)


def kernel(x, edge_index, W1l, b1, W1r, W2l, b2, W2r):
    src = edge_index[0].astype(jnp.int32)
    dst = edge_index[1].astype(jnp.int32)
    zf = jnp.zeros((N_NODES, D_FEAT), jnp.float32)
    ones = jnp.ones((_CHUNK, D_FEAT), jnp.float32)

    agg1, cnt = _sc_pass1(x, src, dst, zf, ones)
    h = _tc1(agg1, cnt, x, W1l, W1r, b1.reshape(1, D_FEAT))
    (agg2,) = _sc_pass2(h, src, dst, zf)
    out = _tc2(agg2, cnt, h, W2l, W2r, b2.reshape(1, D_FEAT))
    return out
